# baseline (speedup 1.0000x reference)
import functools

import jax
import jax.numpy as jnp
from jax import lax
from jax.experimental import pallas as pl
from jax.experimental.pallas import tpu as pltpu
from jax.experimental.pallas import tpu_sc as plsc

NC = 2
NS = 16
NW = NC * NS
CHUNK = 128
F0 = 112
F1 = 48


def _bn_mm_body(x_ref, w_ref, g_ref, bt_ref, y_ref):
    x = x_ref[...]
    mean = jnp.mean(x, axis=0, keepdims=True)
    xc = x - mean
    var = jnp.mean(xc * xc, axis=0, keepdims=True)
    h = xc * lax.rsqrt(var + 1e-5) * g_ref[...] + bt_ref[...]
    h = jnp.maximum(h, 0.0)
    y_ref[...] = jnp.dot(h, w_ref[...], preferred_element_type=jnp.float32)


def _combine_body(x_ref, b_ref, p0_ref, p1_ref, o_ref):
    o_ref[...] = x_ref[...] + b_ref[...] + p0_ref[...] + p1_ref[...]


def _make_scatter(n_acc, d, plane):
    rows_per = n_acc // NS
    h0 = F0 // 2
    h1 = F1 // 2
    mesh = plsc.VectorSubcoreMesh(core_axis_name="c", subcore_axis_name="s")

    @functools.partial(
        pl.kernel,
        out_type=jax.ShapeDtypeStruct((NC, n_acc, d), jnp.float32),
        mesh=mesh,
        scratch_types=[
            pltpu.VMEM_SHARED((n_acc, d), jnp.float32),
            pltpu.VMEM((h0 + 8, CHUNK), jnp.int32),
            pltpu.VMEM((h0, CHUNK), jnp.int32),
            pltpu.VMEM((CHUNK, d), jnp.float32),
            pltpu.VMEM((CHUNK, d), jnp.float32),
            pltpu.SemaphoreType.DMA,
            pltpu.SemaphoreType.DMA,
        ],
    )
    def scatter(y_hbm, src_hbm, dst_hbm, z_hbm, out_hbm,
                acc, sidx, didx, rows0, rows1, sem0, sem1):
        c = lax.axis_index("c")
        s = lax.axis_index("s")
        wid = c * NS + s
        hcp = jnp.where(c == 0, h0, h1)
        pltpu.sync_copy(z_hbm.at[pl.ds(s * rows_per, rows_per)],
                        acc.at[pl.ds(s * rows_per, rows_per)])
        plsc.subcore_barrier()

        for h in range(2):
            off = pl.multiple_of(h * hcp, 8)
            pltpu.sync_copy(src_hbm.at[wid].at[pl.ds(off, h0 + 8)], sidx)
            pltpu.sync_copy(dst_hbm.at[wid].at[pl.ds(off, h0)], didx)
            pltpu.async_copy(y_hbm.at[sidx.at[0]], rows0, sem0)

            def body(i, carry):
                k0 = 2 * i
                k1 = k0 + 1
                pltpu.async_copy(y_hbm.at[sidx.at[k1]], rows1, sem1)
                pltpu.make_async_copy(y_hbm.at[sidx.at[k0]], rows0, sem0).wait()
                pltpu.sync_copy(rows0, acc.at[didx.at[k0]], add=True)
                pltpu.async_copy(y_hbm.at[sidx.at[k0 + 2]], rows0, sem0)
                pltpu.make_async_copy(y_hbm.at[sidx.at[k1]], rows1, sem1).wait()
                pltpu.sync_copy(rows1, acc.at[didx.at[k1]], add=True)
                return carry

            lax.fori_loop(0, hcp // 2, body, 0)
            pltpu.make_async_copy(y_hbm.at[sidx.at[hcp]], rows0, sem0).wait()
        plsc.subcore_barrier()
        pltpu.sync_copy(acc.at[pl.ds(s * rows_per, rows_per)],
                        out_hbm.at[c].at[pl.ds(s * rows_per, rows_per)])

    return scatter


def kernel(x, edge_index, W, b, gamma, beta):
    n, d = x.shape
    e = edge_index.shape[1]

    y = pl.pallas_call(
        _bn_mm_body,
        out_shape=jax.ShapeDtypeStruct((n, d), jnp.float32),
    )(x, W, gamma.reshape(1, d), beta.reshape(1, d))

    n_acc = -(-(n + 1) // (NS * 8)) * (NS * 8)
    tot = (F0 + F1) * NS
    e_pad = tot * CHUNK
    src = edge_index[0].astype(jnp.int32)
    dst = edge_index[1].astype(jnp.int32)
    pad = e_pad - e
    src_c = jnp.concatenate([src, jnp.zeros((pad,), jnp.int32)]).reshape(tot, CHUNK)
    dst_c = jnp.concatenate([dst, jnp.full((pad,), n_acc - 1, jnp.int32)]).reshape(tot, CHUNK)
    plane = F0 + 8
    split = F0 * NS

    def _planes(flat, fill):
        c0 = flat[:split].reshape(NS, F0, CHUNK)
        c1 = flat[split:].reshape(NS, F1, CHUNK)
        pad0 = jnp.full((NS, plane - F0, CHUNK), fill, jnp.int32)
        pad1 = jnp.full((NS, plane - F1, CHUNK), fill, jnp.int32)
        return jnp.concatenate([
            jnp.concatenate([c0, pad0], axis=1),
            jnp.concatenate([c1, pad1], axis=1)], axis=0)

    src_r = _planes(src_c, 0)
    dst_r = _planes(dst_c, n_acc - 1)
    z = jnp.zeros((n_acc, d), jnp.float32)
    p = _make_scatter(n_acc, d, plane)(y, src_r, dst_r, z)[:, :n, :]

    out = pl.pallas_call(
        _combine_body,
        out_shape=jax.ShapeDtypeStruct((n, d), jnp.float32),
    )(x, b.reshape(1, d), p[0], p[1])
    return out


# speedup vs baseline: 1.3787x; 1.3787x over previous
import functools

import jax
import jax.numpy as jnp
from jax import lax
from jax.experimental import pallas as pl
from jax.experimental.pallas import tpu as pltpu
from jax.experimental.pallas import tpu_sc as plsc

NC = 2
NS = 16
NW = NC * NS
CHUNK = 128
F0 = 112
F1 = 48


def _bn_mm_body(x_ref, w_ref, g_ref, bt_ref, y_ref):
    x = x_ref[...]
    mean = jnp.mean(x, axis=0, keepdims=True)
    xc = x - mean
    var = jnp.mean(xc * xc, axis=0, keepdims=True)
    h = xc * lax.rsqrt(var + 1e-5) * g_ref[...] + bt_ref[...]
    h = jnp.maximum(h, 0.0)
    y_ref[...] = jnp.dot(h, w_ref[...], preferred_element_type=jnp.float32)


def _combine_body(x_ref, b_ref, p0_ref, p1_ref, o_ref):
    o_ref[...] = x_ref[...] + b_ref[...] + p0_ref[...] + p1_ref[...]


def _make_scatter(n_acc, d, plane):
    rows_per = n_acc // NS
    h0 = F0 // 2
    h1 = F1 // 2
    mesh = plsc.VectorSubcoreMesh(core_axis_name="c", subcore_axis_name="s")

    @functools.partial(
        pl.kernel,
        out_type=jax.ShapeDtypeStruct((NC, n_acc, d), jnp.float32),
        mesh=mesh,
        scratch_types=[
            pltpu.VMEM_SHARED((n_acc, d), jnp.float32),
            pltpu.VMEM((h0 + 8, CHUNK), jnp.int32),
            pltpu.VMEM((h0, CHUNK), jnp.int32),
            pltpu.VMEM((CHUNK, d), jnp.float32),
            pltpu.VMEM((CHUNK, d), jnp.float32),
            pltpu.SemaphoreType.DMA,
            pltpu.SemaphoreType.DMA,
        ],
    )
    def scatter(y_hbm, src_hbm, dst_hbm, z_hbm, out_hbm,
                acc, sidx, didx, rows0, rows1, sem0, sem1):
        c = lax.axis_index("c")
        s = lax.axis_index("s")
        wid = c * NS + s
        hcp = jnp.where(c == 0, h0, h1)
        pltpu.sync_copy(z_hbm.at[pl.ds(s * rows_per, rows_per)],
                        acc.at[pl.ds(s * rows_per, rows_per)])
        plsc.subcore_barrier()

        for h in range(2):
            off = pl.multiple_of(h * hcp, 8)
            pltpu.sync_copy(src_hbm.at[wid].at[pl.ds(off, h0 + 8)], sidx)
            pltpu.sync_copy(dst_hbm.at[wid].at[pl.ds(off, h0)], didx)
            pltpu.async_copy(y_hbm.at[sidx.at[0]], rows0, sem0)

            def body(i, carry):
                k0 = 2 * i
                k1 = k0 + 1
                pltpu.async_copy(y_hbm.at[sidx.at[k1]], rows1, sem1)
                pltpu.make_async_copy(y_hbm.at[sidx.at[k0]], rows0, sem0).wait()
                pltpu.sync_copy(rows0, acc.at[didx.at[k0]], add=True)
                pltpu.async_copy(y_hbm.at[sidx.at[k0 + 2]], rows0, sem0)
                pltpu.make_async_copy(y_hbm.at[sidx.at[k1]], rows1, sem1).wait()
                pltpu.sync_copy(rows1, acc.at[didx.at[k1]], add=True)
                return carry

            lax.fori_loop(0, hcp // 2, body, 0)
            pltpu.make_async_copy(y_hbm.at[sidx.at[hcp]], rows0, sem0).wait()
        plsc.subcore_barrier()
        pltpu.sync_copy(acc.at[pl.ds(s * rows_per, rows_per)],
                        out_hbm.at[c].at[pl.ds(s * rows_per, rows_per)])

    return scatter


def kernel(x, edge_index, W, b, gamma, beta):
    n, d = x.shape
    e = edge_index.shape[1]

    y = pl.pallas_call(
        _bn_mm_body,
        out_shape=jax.ShapeDtypeStruct((n, d), jnp.float32),
    )(x, W, gamma.reshape(1, d), beta.reshape(1, d))

    n_acc = -(-(n + 1) // (NS * 8)) * (NS * 8)
    tot = (F0 + F1) * NS
    e_pad = tot * CHUNK
    src = edge_index[0].astype(jnp.int32)
    dst = edge_index[1].astype(jnp.int32)
    pad = e_pad - e
    src_c = jnp.concatenate([src, jnp.zeros((pad,), jnp.int32)]).reshape(tot, CHUNK)
    dst_c = jnp.concatenate([dst, jnp.full((pad,), n_acc - 1, jnp.int32)]).reshape(tot, CHUNK)
    plane = F0 + 8
    split = F0 * NS

    def _planes(flat, fill, c1_off=0):
        c0 = flat[:split].reshape(NS, F0, CHUNK)
        c1 = flat[split:].reshape(NS, F1, CHUNK) + c1_off
        pad0 = jnp.full((NS, plane - F0, CHUNK), fill, jnp.int32)
        pad1 = jnp.full((NS, plane - F1, CHUNK), fill, jnp.int32)
        return jnp.concatenate([
            jnp.concatenate([c0, pad0], axis=1),
            jnp.concatenate([c1, pad1], axis=1)], axis=0)

    src_r = _planes(src_c, 0, c1_off=n)
    dst_r = _planes(dst_c, n_acc - 1)
    z = jnp.zeros((n_acc, d), jnp.float32)
    y_rep = jnp.concatenate([y, y], axis=0)
    p = _make_scatter(n_acc, d, plane)(y_rep, src_r, dst_r, z)[:, :n, :]

    out = pl.pallas_call(
        _combine_body,
        out_shape=jax.ShapeDtypeStruct((n, d), jnp.float32),
    )(x, b.reshape(1, d), p[0], p[1])
    return out
